# baseline (speedup 1.0000x reference)
import functools

import jax
import jax.numpy as jnp
from jax import lax
from jax.experimental import pallas as pl
from jax.experimental.pallas import tpu as pltpu
from jax.experimental.pallas import tpu_sc as plsc

N = 10000
D_IN = 128
D_HID = 64
DP = 128
E = 320000

NC = 2
NS = 16
NW = NC * NS
EPW = E // NW
CHUNK = 128
NCHUNK = 80
EPP = NCHUNK * CHUNK
NACC = N + 8
RPS = N // NS
ZROWS = 125
ZROWS_A = 25
ROUT = 624
DEG_W = 128

_mesh = plsc.VectorSubcoreMesh(core_axis_name="c", subcore_axis_name="s")


@functools.partial(
    pl.kernel,
    out_type=jax.ShapeDtypeStruct((NC, N, DEG_W), jnp.float32),
    mesh=_mesh,
    scratch_types=[
        pltpu.VMEM((CHUNK, DEG_W), jnp.float32),
        pltpu.VMEM((NCHUNK, CHUNK), jnp.int32),
        pltpu.VMEM((ZROWS, DEG_W), jnp.float32),
        pltpu.VMEM_SHARED((NACC, DEG_W), jnp.float32),
        pltpu.SemaphoreType.DMA,
    ],
)
def _deg_kernel(dst3_hbm, out_hbm, ones_v, didx_all, zbuf, acc_sh, sem):
    c = lax.axis_index("c")
    s = lax.axis_index("s")
    w = c * NS + s
    one16 = jnp.full((16,), 1.0, jnp.float32)
    zero16 = jnp.zeros((16,), jnp.float32)

    pltpu.async_copy(dst3_hbm.at[w], didx_all, sem)

    @pl.loop(0, CHUNK)
    def _(i):
        @pl.loop(0, DEG_W // 16)
        def _(j):
            ones_v[i, pl.ds(j * 16, 16)] = one16

    @pl.loop(0, ZROWS)
    def _(i):
        @pl.loop(0, DEG_W // 16)
        def _(j):
            zbuf[i, pl.ds(j * 16, 16)] = zero16

    @pl.loop(0, RPS // ZROWS)
    def _(j):
        pltpu.sync_copy(zbuf, acc_sh.at[pl.ds(s * RPS + j * ZROWS, ZROWS)])

    pltpu.make_async_copy(dst3_hbm.at[w], didx_all, sem).wait()
    plsc.subcore_barrier()

    @pl.loop(0, NCHUNK, step=5)
    def _(i):
        descs = [
            pltpu.async_copy(ones_v, acc_sh.at[didx_all.at[i + k]], sem,
                             add=True)
            for k in range(5)
        ]
        for d in descs:
            d.wait()

    plsc.subcore_barrier()
    pltpu.sync_copy(acc_sh.at[pl.ds(s * ROUT, ROUT)],
                    out_hbm.at[c].at[pl.ds(s * ROUT, ROUT)])

    @pl.when(s == NS - 1)
    def _():
        pltpu.sync_copy(acc_sh.at[pl.ds(NS * ROUT, N - NS * ROUT)],
                        out_hbm.at[c].at[pl.ds(NS * ROUT, N - NS * ROUT)])


@functools.partial(
    pl.kernel,
    out_type=jax.ShapeDtypeStruct((NC, N, DP), jnp.float32),
    mesh=_mesh,
    scratch_types=[
        pltpu.VMEM((CHUNK,), jnp.int32),
        pltpu.VMEM((CHUNK,), jnp.int32),
        pltpu.VMEM((NCHUNK, CHUNK), jnp.int32),
        pltpu.VMEM((CHUNK, DP), jnp.float32),
        pltpu.VMEM((CHUNK, DP), jnp.float32),
        pltpu.VMEM((8, DP), jnp.float32),
        pltpu.VMEM_SHARED((NACC, DP), jnp.float32),
        pltpu.SemaphoreType.DMA,
        pltpu.SemaphoreType.DMA,
        pltpu.SemaphoreType.DMA,
        pltpu.SemaphoreType.DMA,
        pltpu.SemaphoreType.DMA,
    ],
)
def _agg_kernel(y_hbm, src2_hbm, dst3_hbm, out_hbm, sbuf_a, sbuf_b, didx_all,
                rows_a, rows_b, zbuf, acc_sh, dsem, isema, isemb, sema, semb):
    c = lax.axis_index("c")
    s = lax.axis_index("s")
    w = c * NS + s
    zero16 = jnp.zeros((16,), jnp.float32)

    pltpu.async_copy(dst3_hbm.at[w], didx_all, dsem)

    def idxload(i, sbuf, isem):
        pltpu.async_copy(src2_hbm.at[pl.ds(w * EPP + i * CHUNK, CHUNK)],
                         sbuf, isem)

    def wait_idx(sbuf, isem):
        pltpu.make_async_copy(src2_hbm.at[pl.ds(0, CHUNK)], sbuf, isem).wait()

    idxload(0, sbuf_a, isema)
    idxload(1, sbuf_b, isemb)

    @pl.loop(0, 8)
    def _(i):
        @pl.loop(0, DP // 16)
        def _(j):
            zbuf[i, pl.ds(j * 16, 16)] = zero16

    @pl.loop(0, 79)
    def _(j):
        pltpu.sync_copy(zbuf, acc_sh.at[pl.ds(s * RPS + j * 8, 8)])

    pltpu.make_async_copy(dst3_hbm.at[w], didx_all, dsem).wait()
    plsc.subcore_barrier()

    def gather(sbuf, buf, sem):
        pltpu.async_copy(y_hbm.at[sbuf], buf, sem)

    def wait_gather(buf, sem):
        pltpu.make_async_copy(y_hbm.at[pl.ds(0, CHUNK)], buf, sem).wait()

    def scat(i, buf):
        pltpu.sync_copy(buf, acc_sh.at[didx_all.at[i]], add=True)

    wait_idx(sbuf_a, isema)
    gather(sbuf_a, rows_a, sema)

    @pl.loop(0, (NCHUNK - 2) // 2)
    def _(j):
        i = 2 * j
        wait_gather(rows_a, sema)
        idxload(i + 2, sbuf_a, isema)
        wait_idx(sbuf_b, isemb)
        gather(sbuf_b, rows_b, semb)
        scat(i, rows_a)
        wait_gather(rows_b, semb)
        idxload(i + 3, sbuf_b, isemb)
        wait_idx(sbuf_a, isema)
        gather(sbuf_a, rows_a, sema)
        scat(i + 1, rows_b)

    wait_gather(rows_a, sema)
    wait_idx(sbuf_b, isemb)
    gather(sbuf_b, rows_b, semb)
    scat(NCHUNK - 2, rows_a)
    wait_gather(rows_b, semb)
    scat(NCHUNK - 1, rows_b)

    plsc.subcore_barrier()
    pltpu.sync_copy(acc_sh.at[pl.ds(s * ROUT, ROUT)],
                    out_hbm.at[c].at[pl.ds(s * ROUT, ROUT)])

    @pl.when(s == NS - 1)
    def _():
        pltpu.sync_copy(acc_sh.at[pl.ds(NS * ROUT, N - NS * ROUT)],
                        out_hbm.at[c].at[pl.ds(NS * ROUT, N - NS * ROUT)])


def _tcmm_body(x_ref, w1_ref, xw_ref):
    xw_ref[...] = jnp.dot(x_ref[...], w1_ref[...],
                          preferred_element_type=jnp.float32)


_tcmm = pl.pallas_call(
    _tcmm_body,
    out_shape=jax.ShapeDtypeStruct((N, DP), jnp.float32),
)


def _tc1_body(xw_ref, dega_ref, y1_ref, dis_ref):
    deg = 1.0 + dega_ref[0, :, 0:1] + dega_ref[1, :, 0:1]
    dis = lax.rsqrt(deg)
    y1_ref[...] = xw_ref[...] * dis
    dis_ref[...] = dis


_tc1 = pl.pallas_call(
    _tc1_body,
    out_shape=(
        jax.ShapeDtypeStruct((N, DP), jnp.float32),
        jax.ShapeDtypeStruct((N, 1), jnp.float32),
    ),
)


def _tc2_body(aggp_ref, y1_ref, dis_ref, b1_ref, w2_ref, y2_ref):
    agg = aggp_ref[0] + aggp_ref[1] + y1_ref[...]
    h = jnp.maximum(dis_ref[...] * agg + b1_ref[...], 0.0)
    y2_ref[...] = (
        jnp.dot(h, w2_ref[...], preferred_element_type=jnp.float32)
        * dis_ref[...]
    )


_tc2 = pl.pallas_call(
    _tc2_body,
    out_shape=jax.ShapeDtypeStruct((N, DP), jnp.float32),
)


def _tc3_body(aggp_ref, y2_ref, dis_ref, b2_ref, wfc_ref, bfc_ref, out_ref):
    agg = aggp_ref[0] + aggp_ref[1] + y2_ref[...]
    h = jnp.maximum(dis_ref[...] * agg + b2_ref[...], 0.0)
    out_ref[...] = (
        jnp.dot(h, wfc_ref[...], preferred_element_type=jnp.float32)
        + bfc_ref[...]
    )


_tc3 = pl.pallas_call(
    _tc3_body,
    out_shape=jax.ShapeDtypeStruct((N, 1), jnp.float32),
)


@jax.jit
def kernel(x, edge_index, W1, b1, W2, b2, Wfc, bfc):
    ei = edge_index.astype(jnp.int32)
    src2 = jnp.pad(ei[0].reshape(NW, EPW), ((0, 0), (0, EPP - EPW))).reshape(-1)
    dst3 = jnp.pad(ei[1].reshape(NW, EPW), ((0, 0), (0, EPP - EPW)),
                   constant_values=N).reshape(NW, NCHUNK, CHUNK)

    w1p = jnp.pad(W1, ((0, 0), (0, DP - D_HID)))
    w2p = jnp.pad(W2, ((0, DP - D_HID), (0, DP - D_HID)))
    b1p = jnp.pad(b1, (0, DP - D_HID))
    b2p = jnp.pad(b2, (0, DP - D_HID))
    wfcp = jnp.pad(Wfc, ((0, DP - D_HID), (0, 0)))

    dega = _deg_kernel(dst3)
    xw1 = _tcmm(x, w1p)
    y1, dis = _tc1(xw1, dega)
    agg1 = _agg_kernel(y1, src2, dst3)
    y2 = _tc2(agg1, y1, dis, b1p, w2p)
    agg2 = _agg_kernel(y2, src2, dst3)
    return _tc3(agg2, y2, dis, b2p, wfcp, bfc)


# speedup vs baseline: 2.2136x; 2.2136x over previous
import functools

import jax
import jax.numpy as jnp
from jax import lax
from jax.experimental import pallas as pl
from jax.experimental.pallas import tpu as pltpu
from jax.experimental.pallas import tpu_sc as plsc

N = 10000
D_IN = 128
D_HID = 64
DP = 128
E = 320000

NC = 2
NS = 16
NW = NC * NS
EPW = E // NW
CHUNK = 80
NCHUNK = EPW // CHUNK
RPS = N // NS
ZROWS = 125
ZROWS_A = 25
ROUT = 624
DEG_W = 128

_mesh = plsc.VectorSubcoreMesh(core_axis_name="c", subcore_axis_name="s")


@functools.partial(
    pl.kernel,
    out_type=jax.ShapeDtypeStruct((NC, N, DEG_W), jnp.float32),
    mesh=_mesh,
    scratch_types=[
        pltpu.VMEM((CHUNK, DEG_W), jnp.float32),
        pltpu.VMEM((NCHUNK, CHUNK), jnp.int32),
        pltpu.VMEM((ZROWS, DEG_W), jnp.float32),
        pltpu.VMEM_SHARED((N, DEG_W), jnp.float32),
        pltpu.SemaphoreType.DMA,
    ],
)
def _deg_kernel(dst3_hbm, out_hbm, ones_v, didx_all, zbuf, acc_sh, sem):
    c = lax.axis_index("c")
    s = lax.axis_index("s")
    w = c * NS + s
    one16 = jnp.full((16,), 1.0, jnp.float32)
    zero16 = jnp.zeros((16,), jnp.float32)

    pltpu.async_copy(dst3_hbm.at[w], didx_all, sem)

    @pl.loop(0, CHUNK)
    def _(i):
        @pl.loop(0, DEG_W // 16)
        def _(j):
            ones_v[i, pl.ds(j * 16, 16)] = one16

    @pl.loop(0, ZROWS)
    def _(i):
        @pl.loop(0, DEG_W // 16)
        def _(j):
            zbuf[i, pl.ds(j * 16, 16)] = zero16

    @pl.loop(0, RPS // ZROWS)
    def _(j):
        pltpu.sync_copy(zbuf, acc_sh.at[pl.ds(s * RPS + j * ZROWS, ZROWS)])

    pltpu.make_async_copy(dst3_hbm.at[w], didx_all, sem).wait()
    plsc.subcore_barrier()

    @pl.loop(0, NCHUNK, step=5)
    def _(i):
        descs = [
            pltpu.async_copy(ones_v, acc_sh.at[didx_all.at[i + k]], sem,
                             add=True)
            for k in range(5)
        ]
        for d in descs:
            d.wait()

    plsc.subcore_barrier()
    pltpu.sync_copy(acc_sh.at[pl.ds(s * ROUT, ROUT)],
                    out_hbm.at[c].at[pl.ds(s * ROUT, ROUT)])

    @pl.when(s == NS - 1)
    def _():
        pltpu.sync_copy(acc_sh.at[pl.ds(NS * ROUT, N - NS * ROUT)],
                        out_hbm.at[c].at[pl.ds(NS * ROUT, N - NS * ROUT)])


@functools.partial(
    pl.kernel,
    out_type=jax.ShapeDtypeStruct((NC, N, DP), jnp.float32),
    mesh=_mesh,
    scratch_types=[
        pltpu.VMEM((EPW,), jnp.int32),
        pltpu.VMEM((NCHUNK, CHUNK), jnp.int32),
        pltpu.VMEM((CHUNK, DP), jnp.float32),
        pltpu.VMEM((CHUNK, DP), jnp.float32),
        pltpu.VMEM((ZROWS_A, DP), jnp.float32),
        pltpu.VMEM_SHARED((N, DP), jnp.float32),
        pltpu.SemaphoreType.DMA,
        pltpu.SemaphoreType.DMA,
        pltpu.SemaphoreType.DMA,
    ],
)
def _agg_kernel(y_hbm, src2_hbm, dst3_hbm, out_hbm, sidx_all, didx_all,
                rows_a, rows_b, zbuf, acc_sh, isem, sema, semb):
    c = lax.axis_index("c")
    s = lax.axis_index("s")
    w = c * NS + s
    zero16 = jnp.zeros((16,), jnp.float32)

    pltpu.async_copy(src2_hbm.at[w], sidx_all, isem)
    pltpu.async_copy(dst3_hbm.at[w], didx_all, isem)

    @pl.loop(0, ZROWS_A)
    def _(i):
        @pl.loop(0, DP // 16)
        def _(j):
            zbuf[i, pl.ds(j * 16, 16)] = zero16

    @pl.loop(0, RPS // ZROWS_A)
    def _(j):
        pltpu.sync_copy(zbuf, acc_sh.at[pl.ds(s * RPS + j * ZROWS_A, ZROWS_A)])

    pltpu.make_async_copy(src2_hbm.at[w], sidx_all, isem).wait()
    pltpu.make_async_copy(dst3_hbm.at[w], didx_all, isem).wait()
    plsc.subcore_barrier()

    def gather(i, buf, sem):
        pltpu.async_copy(y_hbm.at[sidx_all.at[pl.ds(i * CHUNK, CHUNK)]],
                         buf, sem)

    def wait_gather(buf, sem):
        pltpu.make_async_copy(y_hbm.at[pl.ds(0, CHUNK)], buf, sem).wait()

    def scat(i, buf):
        pltpu.sync_copy(buf, acc_sh.at[didx_all.at[i]], add=True)

    gather(0, rows_a, sema)

    @pl.loop(0, (NCHUNK - 1) // 2)
    def _(j):
        i = 2 * j
        wait_gather(rows_a, sema)
        gather(i + 1, rows_b, semb)
        scat(i, rows_a)
        wait_gather(rows_b, semb)
        gather(i + 2, rows_a, sema)
        scat(i + 1, rows_b)

    wait_gather(rows_a, sema)
    scat(NCHUNK - 1, rows_a)

    plsc.subcore_barrier()
    pltpu.sync_copy(acc_sh.at[pl.ds(s * ROUT, ROUT)],
                    out_hbm.at[c].at[pl.ds(s * ROUT, ROUT)])

    @pl.when(s == NS - 1)
    def _():
        pltpu.sync_copy(acc_sh.at[pl.ds(NS * ROUT, N - NS * ROUT)],
                        out_hbm.at[c].at[pl.ds(NS * ROUT, N - NS * ROUT)])


def _tcmm_body(x_ref, w1_ref, xw_ref):
    xw_ref[...] = jnp.dot(x_ref[...], w1_ref[...],
                          preferred_element_type=jnp.float32)


_tcmm = pl.pallas_call(
    _tcmm_body,
    out_shape=jax.ShapeDtypeStruct((N, DP), jnp.float32),
)


def _tc1_body(xw_ref, dega_ref, y1_ref, dis_ref):
    deg = 1.0 + dega_ref[0, :, 0:1] + dega_ref[1, :, 0:1]
    dis = lax.rsqrt(deg)
    y1_ref[...] = xw_ref[...] * dis
    dis_ref[...] = dis


_tc1 = pl.pallas_call(
    _tc1_body,
    out_shape=(
        jax.ShapeDtypeStruct((N, DP), jnp.float32),
        jax.ShapeDtypeStruct((N, 1), jnp.float32),
    ),
)


def _tc2_body(aggp_ref, y1_ref, dis_ref, b1_ref, w2_ref, y2_ref):
    agg = aggp_ref[0] + aggp_ref[1] + y1_ref[...]
    h = jnp.maximum(dis_ref[...] * agg + b1_ref[...], 0.0)
    y2_ref[...] = (
        jnp.dot(h, w2_ref[...], preferred_element_type=jnp.float32)
        * dis_ref[...]
    )


_tc2 = pl.pallas_call(
    _tc2_body,
    out_shape=jax.ShapeDtypeStruct((N, DP), jnp.float32),
)


def _tc3_body(aggp_ref, y2_ref, dis_ref, b2_ref, wfc_ref, bfc_ref, out_ref):
    agg = aggp_ref[0] + aggp_ref[1] + y2_ref[...]
    h = jnp.maximum(dis_ref[...] * agg + b2_ref[...], 0.0)
    out_ref[...] = (
        jnp.dot(h, wfc_ref[...], preferred_element_type=jnp.float32)
        + bfc_ref[...]
    )


_tc3 = pl.pallas_call(
    _tc3_body,
    out_shape=jax.ShapeDtypeStruct((N, 1), jnp.float32),
)


@jax.jit
def kernel(x, edge_index, W1, b1, W2, b2, Wfc, bfc):
    ei = edge_index.astype(jnp.int32)
    src2 = ei[0].reshape(NW, EPW)
    dst3 = ei[1].reshape(NW, NCHUNK, CHUNK)

    w1p = jnp.pad(W1, ((0, 0), (0, DP - D_HID)))
    w2p = jnp.pad(W2, ((0, DP - D_HID), (0, DP - D_HID)))
    b1p = jnp.pad(b1, (0, DP - D_HID))
    b2p = jnp.pad(b2, (0, DP - D_HID))
    wfcp = jnp.pad(Wfc, ((0, DP - D_HID), (0, 0)))

    dega = _deg_kernel(dst3)
    xw1 = _tcmm(x, w1p)
    y1, dis = _tc1(xw1, dega)
    agg1 = _agg_kernel(y1, src2, dst3)
    y2 = _tc2(agg1, y1, dis, b1p, w2p)
    agg2 = _agg_kernel(y2, src2, dst3)
    return _tc3(agg2, y2, dis, b2p, wfcp, bfc)


# speedup vs baseline: 2.6597x; 1.2015x over previous
import functools

import jax
import jax.numpy as jnp
from jax import lax
from jax.experimental import pallas as pl
from jax.experimental.pallas import tpu as pltpu
from jax.experimental.pallas import tpu_sc as plsc

N = 10000
D_IN = 128
D_HID = 64
DP = 128
E = 320000

NC = 2
NS = 16
NW = NC * NS
EPW = E // NW
CHUNK = 80
NCHUNK = EPW // CHUNK
RPS = N // NS
ZROWS = 125
ZROWS_A = 25
ROUT = 624
DEG_W = 128

_mesh = plsc.VectorSubcoreMesh(core_axis_name="c", subcore_axis_name="s")


@functools.partial(
    pl.kernel,
    out_type=jax.ShapeDtypeStruct((NC, N, DEG_W), jnp.float32),
    mesh=_mesh,
    scratch_types=[
        pltpu.VMEM((CHUNK, DEG_W), jnp.float32),
        pltpu.VMEM((NCHUNK, CHUNK), jnp.int32),
        pltpu.VMEM((ZROWS, DEG_W), jnp.float32),
        pltpu.VMEM_SHARED((N, DEG_W), jnp.float32),
        pltpu.SemaphoreType.DMA,
    ],
)
def _deg_kernel(dst3_hbm, out_hbm, ones_v, didx_all, zbuf, acc_sh, sem):
    c = lax.axis_index("c")
    s = lax.axis_index("s")
    w = c * NS + s
    one16 = jnp.full((16,), 1.0, jnp.float32)
    zero16 = jnp.zeros((16,), jnp.float32)

    pltpu.async_copy(dst3_hbm.at[w], didx_all, sem)

    @pl.loop(0, CHUNK)
    def _(i):
        @pl.loop(0, DEG_W // 16)
        def _(j):
            ones_v[i, pl.ds(j * 16, 16)] = one16

    @pl.loop(0, ZROWS)
    def _(i):
        @pl.loop(0, DEG_W // 16)
        def _(j):
            zbuf[i, pl.ds(j * 16, 16)] = zero16

    @pl.loop(0, RPS // ZROWS)
    def _(j):
        pltpu.sync_copy(zbuf, acc_sh.at[pl.ds(s * RPS + j * ZROWS, ZROWS)])

    pltpu.make_async_copy(dst3_hbm.at[w], didx_all, sem).wait()
    plsc.subcore_barrier()

    @pl.loop(0, NCHUNK, step=5)
    def _(i):
        descs = [
            pltpu.async_copy(ones_v, acc_sh.at[didx_all.at[i + k]], sem,
                             add=True)
            for k in range(5)
        ]
        for d in descs:
            d.wait()

    plsc.subcore_barrier()
    pltpu.sync_copy(acc_sh.at[pl.ds(s * ROUT, ROUT)],
                    out_hbm.at[c].at[pl.ds(s * ROUT, ROUT)])

    @pl.when(s == NS - 1)
    def _():
        pltpu.sync_copy(acc_sh.at[pl.ds(NS * ROUT, N - NS * ROUT)],
                        out_hbm.at[c].at[pl.ds(NS * ROUT, N - NS * ROUT)])


@functools.partial(
    pl.kernel,
    out_type=jax.ShapeDtypeStruct((NC, N, DP), jnp.float32),
    mesh=_mesh,
    scratch_types=[
        pltpu.VMEM((EPW,), jnp.int32),
        pltpu.VMEM((NCHUNK, CHUNK), jnp.int32),
        pltpu.VMEM((CHUNK, DP), jnp.float32),
        pltpu.VMEM((CHUNK, DP), jnp.float32),
        pltpu.VMEM((ZROWS_A, DP), jnp.float32),
        pltpu.VMEM_SHARED((N, DP), jnp.float32),
        pltpu.SemaphoreType.DMA,
        pltpu.SemaphoreType.DMA,
        pltpu.SemaphoreType.DMA,
    ],
)
def _agg_kernel(y_hbm, src2_hbm, dst3_hbm, out_hbm, sidx_all, didx_all,
                rows_a, rows_b, zbuf, acc_sh, isem, sema, semb):
    c = lax.axis_index("c")
    s = lax.axis_index("s")
    w = c * NS + s
    zero16 = jnp.zeros((16,), jnp.float32)

    pltpu.async_copy(src2_hbm.at[w], sidx_all, isem)
    pltpu.async_copy(dst3_hbm.at[w], didx_all, isem)

    @pl.loop(0, ZROWS_A)
    def _(i):
        @pl.loop(0, DP // 16)
        def _(j):
            zbuf[i, pl.ds(j * 16, 16)] = zero16

    @pl.loop(0, RPS // ZROWS_A)
    def _(j):
        pltpu.sync_copy(zbuf, acc_sh.at[pl.ds(s * RPS + j * ZROWS_A, ZROWS_A)])

    pltpu.make_async_copy(src2_hbm.at[w], sidx_all, isem).wait()
    pltpu.make_async_copy(dst3_hbm.at[w], didx_all, isem).wait()
    plsc.subcore_barrier()

    def gather(i, buf, sem):
        pltpu.async_copy(y_hbm.at[sidx_all.at[pl.ds(i * CHUNK, CHUNK)]],
                         buf, sem)

    def wait_gather(buf, sem):
        pltpu.make_async_copy(y_hbm.at[pl.ds(0, CHUNK)], buf, sem).wait()

    def scat(i, buf):
        pltpu.sync_copy(buf, acc_sh.at[didx_all.at[i]], add=True)

    gather(0, rows_a, sema)
    gather(1, rows_b, semb)

    @pl.loop(0, (NCHUNK - 3) // 2)
    def _(j):
        i = 2 * j
        wait_gather(rows_a, sema)
        scat(i, rows_a)
        gather(i + 2, rows_a, sema)
        wait_gather(rows_b, semb)
        scat(i + 1, rows_b)
        gather(i + 3, rows_b, semb)

    wait_gather(rows_a, sema)
    scat(NCHUNK - 3, rows_a)
    gather(NCHUNK - 1, rows_a, sema)
    wait_gather(rows_b, semb)
    scat(NCHUNK - 2, rows_b)
    wait_gather(rows_a, sema)
    scat(NCHUNK - 1, rows_a)

    plsc.subcore_barrier()
    pltpu.sync_copy(acc_sh.at[pl.ds(s * ROUT, ROUT)],
                    out_hbm.at[c].at[pl.ds(s * ROUT, ROUT)])

    @pl.when(s == NS - 1)
    def _():
        pltpu.sync_copy(acc_sh.at[pl.ds(NS * ROUT, N - NS * ROUT)],
                        out_hbm.at[c].at[pl.ds(NS * ROUT, N - NS * ROUT)])


def _tcmm_body(x_ref, w1_ref, xw_ref):
    xw_ref[...] = jnp.dot(x_ref[...], w1_ref[...],
                          preferred_element_type=jnp.float32)


_tcmm = pl.pallas_call(
    _tcmm_body,
    out_shape=jax.ShapeDtypeStruct((N, DP), jnp.float32),
)


def _tc1_body(xw_ref, dega_ref, y1_ref, dis_ref):
    deg = 1.0 + dega_ref[0, :, 0:1] + dega_ref[1, :, 0:1]
    dis = lax.rsqrt(deg)
    y1_ref[...] = xw_ref[...] * dis
    dis_ref[...] = dis


_tc1 = pl.pallas_call(
    _tc1_body,
    out_shape=(
        jax.ShapeDtypeStruct((N, DP), jnp.float32),
        jax.ShapeDtypeStruct((N, 1), jnp.float32),
    ),
)


def _tc2_body(aggp_ref, y1_ref, dis_ref, b1_ref, w2_ref, y2_ref):
    agg = aggp_ref[0] + aggp_ref[1] + y1_ref[...]
    h = jnp.maximum(dis_ref[...] * agg + b1_ref[...], 0.0)
    y2_ref[...] = (
        jnp.dot(h, w2_ref[...], preferred_element_type=jnp.float32)
        * dis_ref[...]
    )


_tc2 = pl.pallas_call(
    _tc2_body,
    out_shape=jax.ShapeDtypeStruct((N, DP), jnp.float32),
)


def _tc3_body(aggp_ref, y2_ref, dis_ref, b2_ref, wfc_ref, bfc_ref, out_ref):
    agg = aggp_ref[0] + aggp_ref[1] + y2_ref[...]
    h = jnp.maximum(dis_ref[...] * agg + b2_ref[...], 0.0)
    out_ref[...] = (
        jnp.dot(h, wfc_ref[...], preferred_element_type=jnp.float32)
        + bfc_ref[...]
    )


_tc3 = pl.pallas_call(
    _tc3_body,
    out_shape=jax.ShapeDtypeStruct((N, 1), jnp.float32),
)


@jax.jit
def kernel(x, edge_index, W1, b1, W2, b2, Wfc, bfc):
    ei = edge_index.astype(jnp.int32)
    src2 = ei[0].reshape(NW, EPW)
    dst3 = ei[1].reshape(NW, NCHUNK, CHUNK)

    w1p = jnp.pad(W1, ((0, 0), (0, DP - D_HID)))
    w2p = jnp.pad(W2, ((0, DP - D_HID), (0, DP - D_HID)))
    b1p = jnp.pad(b1, (0, DP - D_HID))
    b2p = jnp.pad(b2, (0, DP - D_HID))
    wfcp = jnp.pad(Wfc, ((0, DP - D_HID), (0, 0)))

    dega = _deg_kernel(dst3)
    xw1 = _tcmm(x, w1p)
    y1, dis = _tc1(xw1, dega)
    agg1 = _agg_kernel(y1, src2, dst3)
    y2 = _tc2(agg1, y1, dis, b1p, w2p)
    agg2 = _agg_kernel(y2, src2, dst3)
    return _tc3(agg2, y2, dis, b2p, wfcp, bfc)


# speedup vs baseline: 2.6607x; 1.0004x over previous
import functools

import jax
import jax.numpy as jnp
from jax import lax
from jax.experimental import pallas as pl
from jax.experimental.pallas import tpu as pltpu
from jax.experimental.pallas import tpu_sc as plsc

N = 10000
D_IN = 128
D_HID = 64
DP = 128
E = 320000

NC = 2
NS = 16
NW = NC * NS
EPW = E // NW
CHUNK = 80
NCHUNK = EPW // CHUNK
RPS = N // NS
ZROWS = 125
ZROWS_A = 25
ROUT = 624
DEG_W = 128

_mesh = plsc.VectorSubcoreMesh(core_axis_name="c", subcore_axis_name="s")


@functools.partial(
    pl.kernel,
    out_type=jax.ShapeDtypeStruct((NC, N, DEG_W), jnp.float32),
    mesh=_mesh,
    scratch_types=[
        pltpu.VMEM((CHUNK, DEG_W), jnp.float32),
        pltpu.VMEM((NCHUNK, CHUNK), jnp.int32),
        pltpu.VMEM((ZROWS, DEG_W), jnp.float32),
        pltpu.VMEM_SHARED((N, DEG_W), jnp.float32),
        pltpu.SemaphoreType.DMA,
    ],
)
def _deg_kernel(dst3_hbm, out_hbm, ones_v, didx_all, zbuf, acc_sh, sem):
    c = lax.axis_index("c")
    s = lax.axis_index("s")
    w = c * NS + s
    one16 = jnp.full((16,), 1.0, jnp.float32)
    zero16 = jnp.zeros((16,), jnp.float32)

    pltpu.async_copy(dst3_hbm.at[w], didx_all, sem)

    @pl.loop(0, CHUNK)
    def _(i):
        @pl.loop(0, DEG_W // 16)
        def _(j):
            ones_v[i, pl.ds(j * 16, 16)] = one16

    @pl.loop(0, ZROWS)
    def _(i):
        @pl.loop(0, DEG_W // 16)
        def _(j):
            zbuf[i, pl.ds(j * 16, 16)] = zero16

    @pl.loop(0, RPS // ZROWS)
    def _(j):
        pltpu.sync_copy(zbuf, acc_sh.at[pl.ds(s * RPS + j * ZROWS, ZROWS)])

    pltpu.make_async_copy(dst3_hbm.at[w], didx_all, sem).wait()
    plsc.subcore_barrier()

    for k in range(5):
        pltpu.async_copy(ones_v, acc_sh.at[didx_all.at[k]], sem, add=True)

    @pl.loop(5, NCHUNK)
    def _(i):
        pltpu.async_copy(ones_v, acc_sh.at[didx_all.at[i]], sem, add=True)
        pltpu.make_async_copy(ones_v, acc_sh.at[didx_all.at[0]], sem).wait()

    for k in range(5):
        pltpu.make_async_copy(ones_v, acc_sh.at[didx_all.at[0]], sem).wait()

    plsc.subcore_barrier()
    pltpu.sync_copy(acc_sh.at[pl.ds(s * ROUT, ROUT)],
                    out_hbm.at[c].at[pl.ds(s * ROUT, ROUT)])

    @pl.when(s == NS - 1)
    def _():
        pltpu.sync_copy(acc_sh.at[pl.ds(NS * ROUT, N - NS * ROUT)],
                        out_hbm.at[c].at[pl.ds(NS * ROUT, N - NS * ROUT)])


@functools.partial(
    pl.kernel,
    out_type=jax.ShapeDtypeStruct((NC, N, DP), jnp.float32),
    mesh=_mesh,
    scratch_types=[
        pltpu.VMEM((EPW,), jnp.int32),
        pltpu.VMEM((NCHUNK, CHUNK), jnp.int32),
        pltpu.VMEM((CHUNK, DP), jnp.float32),
        pltpu.VMEM((CHUNK, DP), jnp.float32),
        pltpu.VMEM((ZROWS_A, DP), jnp.float32),
        pltpu.VMEM_SHARED((N, DP), jnp.float32),
        pltpu.SemaphoreType.DMA,
        pltpu.SemaphoreType.DMA,
        pltpu.SemaphoreType.DMA,
    ],
)
def _agg_kernel(y_hbm, src2_hbm, dst3_hbm, out_hbm, sidx_all, didx_all,
                rows_a, rows_b, zbuf, acc_sh, isem, sema, semb):
    c = lax.axis_index("c")
    s = lax.axis_index("s")
    w = c * NS + s
    zero16 = jnp.zeros((16,), jnp.float32)

    pltpu.async_copy(src2_hbm.at[w], sidx_all, isem)
    pltpu.async_copy(dst3_hbm.at[w], didx_all, isem)

    @pl.loop(0, ZROWS_A)
    def _(i):
        @pl.loop(0, DP // 16)
        def _(j):
            zbuf[i, pl.ds(j * 16, 16)] = zero16

    @pl.loop(0, RPS // ZROWS_A)
    def _(j):
        pltpu.sync_copy(zbuf, acc_sh.at[pl.ds(s * RPS + j * ZROWS_A, ZROWS_A)])

    pltpu.make_async_copy(src2_hbm.at[w], sidx_all, isem).wait()
    pltpu.make_async_copy(dst3_hbm.at[w], didx_all, isem).wait()
    plsc.subcore_barrier()

    def gather(i, buf, sem):
        pltpu.async_copy(y_hbm.at[sidx_all.at[pl.ds(i * CHUNK, CHUNK)]],
                         buf, sem)

    def wait_gather(buf, sem):
        pltpu.make_async_copy(y_hbm.at[pl.ds(0, CHUNK)], buf, sem).wait()

    def scat(i, buf):
        pltpu.sync_copy(buf, acc_sh.at[didx_all.at[i]], add=True)

    gather(0, rows_a, sema)
    gather(1, rows_b, semb)

    @pl.loop(0, (NCHUNK - 3) // 2)
    def _(j):
        i = 2 * j
        wait_gather(rows_a, sema)
        scat(i, rows_a)
        gather(i + 2, rows_a, sema)
        wait_gather(rows_b, semb)
        scat(i + 1, rows_b)
        gather(i + 3, rows_b, semb)

    wait_gather(rows_a, sema)
    scat(NCHUNK - 3, rows_a)
    gather(NCHUNK - 1, rows_a, sema)
    wait_gather(rows_b, semb)
    scat(NCHUNK - 2, rows_b)
    wait_gather(rows_a, sema)
    scat(NCHUNK - 1, rows_a)

    plsc.subcore_barrier()
    pltpu.sync_copy(acc_sh.at[pl.ds(s * ROUT, ROUT)],
                    out_hbm.at[c].at[pl.ds(s * ROUT, ROUT)])

    @pl.when(s == NS - 1)
    def _():
        pltpu.sync_copy(acc_sh.at[pl.ds(NS * ROUT, N - NS * ROUT)],
                        out_hbm.at[c].at[pl.ds(NS * ROUT, N - NS * ROUT)])


def _tc1_body(x_ref, w1_ref, dega_ref, y1_ref, dis_ref):
    deg = 1.0 + dega_ref[0, :, 0:1] + dega_ref[1, :, 0:1]
    dis = lax.rsqrt(deg)
    xw = jnp.dot(x_ref[...], w1_ref[...], preferred_element_type=jnp.float32)
    y1_ref[...] = xw * dis
    dis_ref[...] = dis


_tc1 = pl.pallas_call(
    _tc1_body,
    out_shape=(
        jax.ShapeDtypeStruct((N, DP), jnp.float32),
        jax.ShapeDtypeStruct((N, 1), jnp.float32),
    ),
)


def _tc2_body(aggp_ref, y1_ref, dis_ref, b1_ref, w2_ref, y2_ref):
    agg = aggp_ref[0] + aggp_ref[1] + y1_ref[...]
    h = jnp.maximum(dis_ref[...] * agg + b1_ref[...], 0.0)
    y2_ref[...] = (
        jnp.dot(h, w2_ref[...], preferred_element_type=jnp.float32)
        * dis_ref[...]
    )


_tc2 = pl.pallas_call(
    _tc2_body,
    out_shape=jax.ShapeDtypeStruct((N, DP), jnp.float32),
)


def _tc3_body(aggp_ref, y2_ref, dis_ref, b2_ref, wfc_ref, bfc_ref, out_ref):
    agg = aggp_ref[0] + aggp_ref[1] + y2_ref[...]
    h = jnp.maximum(dis_ref[...] * agg + b2_ref[...], 0.0)
    out_ref[...] = (
        jnp.dot(h, wfc_ref[...], preferred_element_type=jnp.float32)
        + bfc_ref[...]
    )


_tc3 = pl.pallas_call(
    _tc3_body,
    out_shape=jax.ShapeDtypeStruct((N, 1), jnp.float32),
)


@jax.jit
def kernel(x, edge_index, W1, b1, W2, b2, Wfc, bfc):
    ei = edge_index.astype(jnp.int32)
    src2 = ei[0].reshape(NW, EPW)
    dst3 = ei[1].reshape(NW, NCHUNK, CHUNK)

    w1p = jnp.pad(W1, ((0, 0), (0, DP - D_HID)))
    w2p = jnp.pad(W2, ((0, DP - D_HID), (0, DP - D_HID)))
    b1p = jnp.pad(b1, (0, DP - D_HID))
    b2p = jnp.pad(b2, (0, DP - D_HID))
    wfcp = jnp.pad(Wfc, ((0, DP - D_HID), (0, 0)))

    dega = _deg_kernel(dst3)
    y1, dis = _tc1(x, w1p, dega)
    agg1 = _agg_kernel(y1, src2, dst3)
    y2 = _tc2(agg1, y1, dis, b1p, w2p)
    agg2 = _agg_kernel(y2, src2, dst3)
    return _tc3(agg2, y2, dis, b2p, wfcp, bfc)
